# baseline (speedup 1.0000x reference)
import functools

import jax
import jax.numpy as jnp
from jax.experimental import pallas as pl
from jax.experimental.pallas import tpu as pltpu

_B = None
_N_ATOMS = 21
_D = 30


def _silu(x):
    h = 0.5 * x
    return h + h * jnp.tanh(h)


def _hessian_kernel(
    s_ref, v_ref, rz_ref,
    wmix1_ref, w1a_ref, b1a_ref, w2a_ref, b2a_ref,
    wmix2_ref, w1b_ref, b1b_ref, w2b_ref, b2b_ref,
    wp1_ref, bp1_ref, wp2_ref, bp2_ref,
    wf1_ref, bf1_ref, wf2_ref, bf2_ref,
    wg1_ref, bg1_ref, wg2x4_ref, bg2x4_ref, e4_ref,
    out_ref,
    *, g_mols, n, d, pair_k,
):
    a_rows = g_mols * n

    s = s_ref[...]

    def finish_block(s_in, mx, my, mz, w1, b1, w2, b2, sact):
        vVx, vWx = mx[:, :d], mx[:, d:]
        vVy, vWy = my[:, :d], my[:, d:]
        vVz, vWz = mz[:, :d], mz[:, d:]
        vVn = jnp.sqrt(vVx * vVx + vVy * vVy + vVz * vVz)
        h = jnp.dot(s_in, w1[:d, :], preferred_element_type=jnp.float32)
        h += jnp.dot(vVn, w1[d:, :], preferred_element_type=jnp.float32)
        h = _silu(h + b1)
        x = jnp.dot(h, w2, preferred_element_type=jnp.float32) + b2
        s_out = x[:, :d]
        gate = x[:, d:]
        if sact:
            s_out = _silu(s_out)
        return s_out, gate * vWx, gate * vWy, gate * vWz

    mcat = jnp.dot(v_ref[...], wmix1_ref[...],
                   preferred_element_type=jnp.float32)
    s1, v1x, v1y, v1z = finish_block(
        s, mcat[:, :2 * d], mcat[:, 2 * d:4 * d], mcat[:, 4 * d:],
        w1a_ref[...], b1a_ref[...], w2a_ref[...], b2a_ref[...], True)
    wmix2 = wmix2_ref[...]
    s2, v2x, v2y, v2z = finish_block(
        s1,
        jnp.dot(v1x, wmix2, preferred_element_type=jnp.float32),
        jnp.dot(v1y, wmix2, preferred_element_type=jnp.float32),
        jnp.dot(v1z, wmix2, preferred_element_type=jnp.float32),
        w1b_ref[...], b1b_ref[...], w2b_ref[...], b2b_ref[...], False)

    ph = _silu(jnp.dot(rz_ref[...], wp1_ref[...],
                       preferred_element_type=jnp.float32) + bp1_ref[...])
    pos = jnp.dot(ph, wp2_ref[...], preferred_element_type=jnp.float32) + bp2_ref[...]

    wf1 = wf1_ref[...]
    fp = jnp.dot(s2, wf1[0:d, :], preferred_element_type=jnp.float32)
    fp += jnp.dot(v2x, wf1[d:2 * d, :], preferred_element_type=jnp.float32)
    fp += jnp.dot(v2y, wf1[2 * d:3 * d, :], preferred_element_type=jnp.float32)
    fp += jnp.dot(v2z, wf1[3 * d:4 * d, :], preferred_element_type=jnp.float32)
    fp += jnp.dot(pos, wf1[4 * d:, :], preferred_element_type=jnp.float32)
    f = jnp.dot(_silu(fp + bf1_ref[...]), wf2_ref[...],
                preferred_element_type=jnp.float32) + bf2_ref[...]

    wg1 = wg1_ref[...]
    nf = wf2_ref.shape[1]
    a1 = jnp.dot(f, wg1[:nf, :], preferred_element_type=jnp.float32) + bg1_ref[...]
    a2 = jnp.dot(f, wg1[nf:, :], preferred_element_type=jnp.float32)

    K = pair_k
    a2_3 = a2.reshape(g_mols, n, d)
    wg2k = wg2x4_ref[...]
    bg2k = bg2x4_ref[...]
    a1t = jnp.dot(a1, e4_ref[...],
                  preferred_element_type=jnp.float32).reshape(g_mols, n, K * d)
    for grp in range((n + K - 1) // K):
        j0 = K * grp
        js = [min(j0 + t, n - 1) for t in range(K)]
        a2g = jnp.concatenate(
            [a2_3[:, j:j + 1, :] for j in js], axis=2)
        h = (a1t + a2g).reshape(a_rows, K * d)
        sil = h + h * jnp.tanh(h)
        g4 = jnp.dot(sil, wg2k, preferred_element_type=jnp.float32) + bg2k
        w = 3 * min(K, n - j0)
        for a in range(3):
            c0 = 3 * n * a + 3 * j0
            out_ref[:, c0:c0 + w] = g4[:, 3 * K * a:3 * K * a + w]


@functools.partial(jax.jit, static_argnames=("interpret",))
def _run(R, s_rep, v_rep, params, Z, interpret=False):
    p = params
    N = R.shape[0]
    n = _N_ATOMS
    d = _D
    B = N // n
    G = 64
    K = 7
    A = G * n

    v_flat = v_rep.reshape(N, 3 * d)
    onehot = (Z[:, None] == jnp.array([1, 6, 8], dtype=Z.dtype)[None, :]).astype(jnp.float32)
    rz = jnp.concatenate([R, onehot], axis=1)

    def row(b):
        return b.reshape(1, -1)

    weights = [
        jax.scipy.linalg.block_diag(*([p['Wmix1']] * 3)),
        p['W1a'], row(p['b1a']), p['W2a'], row(p['b2a']),
        p['Wmix2'], p['W1b'], row(p['b1b']), p['W2b'], row(p['b2b']),
        p['Wp1'], row(p['bp1']), p['Wp2'], row(p['bp2']),
        p['Wf1'], row(p['bf1']), p['Wf2'], row(p['bf2']),
        0.5 * p['Wg1'], 0.5 * row(p['bg1']),
        jax.scipy.linalg.block_diag(*([p['Wg2']] * K)).reshape(K * d, K, 3, 3)
            .transpose(0, 2, 1, 3).reshape(K * d, 9 * K),
        row(p['bg2']).reshape(1, 1, 3, 3).repeat(K, axis=1)
            .transpose(0, 2, 1, 3).reshape(1, 9 * K),
        jnp.concatenate([jnp.eye(d, dtype=jnp.float32)] * K, axis=1),
    ]

    full = [pl.BlockSpec(w.shape, lambda g, nd=w.ndim: (0,) * nd) for w in weights]

    out = pl.pallas_call(
        functools.partial(_hessian_kernel, g_mols=G, n=n, d=d, pair_k=K),
        grid=(B // G,),
        in_specs=[
            pl.BlockSpec((A, d), lambda g: (g, 0)),
            pl.BlockSpec((A, 3 * d), lambda g: (g, 0)),
            pl.BlockSpec((A, 6), lambda g: (g, 0)),
        ] + full,
        out_specs=pl.BlockSpec((A, 9 * n), lambda g: (g, 0)),
        out_shape=jax.ShapeDtypeStruct((N, 9 * n), jnp.float32),
        interpret=interpret,
    )(s_rep, v_flat, rz, *weights)

    return out.reshape(-1, 27)


def kernel(R, scalar_representation, vector_representation, params, Z, n_atoms):
    del n_atoms
    return _run(R, scalar_representation, vector_representation, params, Z)


# speedup vs baseline: 1.0583x; 1.0583x over previous
import functools

import jax
import jax.numpy as jnp
from jax.experimental import pallas as pl
from jax.experimental.pallas import tpu as pltpu

_B = None
_N_ATOMS = 21
_D = 30


def _silu(x):
    h = 0.5 * x
    return h + h * jnp.tanh(h)


def _hessian_kernel(
    s_ref, v_ref, rz_ref,
    wmix1_ref, w1a_ref, b1a_ref, w2a_ref, b2a_ref,
    wmix2_ref, w1b_ref, b1b_ref, w2b_ref, b2b_ref,
    wp1_ref, bp1_ref, wp2_ref, bp2_ref,
    wf1_ref, bf1_ref, wf2_ref, bf2_ref,
    wg1_ref, bg1_ref, wg2x4_ref, bg2x4_ref, e4_ref,
    out_ref,
    *, g_mols, n, d, pair_k,
):
    a_rows = g_mols * n

    s = s_ref[...]

    def finish_block(s_in, mx, my, mz, w1, b1, w2, b2, sact):
        vVx, vWx = mx[:, :d], mx[:, d:]
        vVy, vWy = my[:, :d], my[:, d:]
        vVz, vWz = mz[:, :d], mz[:, d:]
        vVn = jnp.sqrt(vVx * vVx + vVy * vVy + vVz * vVz)
        h = jnp.dot(s_in, w1[:d, :], preferred_element_type=jnp.float32)
        h += jnp.dot(vVn, w1[d:, :], preferred_element_type=jnp.float32)
        h = _silu(h + b1)
        x = jnp.dot(h, w2, preferred_element_type=jnp.float32) + b2
        s_out = x[:, :d]
        gate = x[:, d:]
        if sact:
            s_out = _silu(s_out)
        return s_out, gate * vWx, gate * vWy, gate * vWz

    mcat = jnp.dot(v_ref[...], wmix1_ref[...],
                   preferred_element_type=jnp.float32)
    s1, v1x, v1y, v1z = finish_block(
        s, mcat[:, :2 * d], mcat[:, 2 * d:4 * d], mcat[:, 4 * d:],
        w1a_ref[...], b1a_ref[...], w2a_ref[...], b2a_ref[...], True)
    wmix2 = wmix2_ref[...]
    s2, v2x, v2y, v2z = finish_block(
        s1,
        jnp.dot(v1x, wmix2, preferred_element_type=jnp.float32),
        jnp.dot(v1y, wmix2, preferred_element_type=jnp.float32),
        jnp.dot(v1z, wmix2, preferred_element_type=jnp.float32),
        w1b_ref[...], b1b_ref[...], w2b_ref[...], b2b_ref[...], False)

    ph = _silu(jnp.dot(rz_ref[...], wp1_ref[...],
                       preferred_element_type=jnp.float32) + bp1_ref[...])
    pos = jnp.dot(ph, wp2_ref[...], preferred_element_type=jnp.float32) + bp2_ref[...]

    wf1 = wf1_ref[...]
    fp = jnp.dot(s2, wf1[0:d, :], preferred_element_type=jnp.float32)
    fp += jnp.dot(v2x, wf1[d:2 * d, :], preferred_element_type=jnp.float32)
    fp += jnp.dot(v2y, wf1[2 * d:3 * d, :], preferred_element_type=jnp.float32)
    fp += jnp.dot(v2z, wf1[3 * d:4 * d, :], preferred_element_type=jnp.float32)
    fp += jnp.dot(pos, wf1[4 * d:, :], preferred_element_type=jnp.float32)
    f = jnp.dot(_silu(fp + bf1_ref[...]), wf2_ref[...],
                preferred_element_type=jnp.float32) + bf2_ref[...]

    wg1 = wg1_ref[...]
    nf = wf2_ref.shape[1]
    a1 = jnp.dot(f, wg1[:nf, :], preferred_element_type=jnp.float32) + bg1_ref[...]
    a2 = jnp.dot(f, wg1[nf:, :], preferred_element_type=jnp.float32)

    K = pair_k
    a2_3 = a2.reshape(g_mols, n, d)
    wg2k = wg2x4_ref[...]
    bg2k = bg2x4_ref[...]
    a1t = jnp.dot(a1, e4_ref[...],
                  preferred_element_type=jnp.float32).reshape(g_mols, n, K * d)
    for grp in range((n + K - 1) // K):
        j0 = K * grp
        js = [min(j0 + t, n - 1) for t in range(K)]
        a2g = jnp.concatenate(
            [a2_3[:, j:j + 1, :] for j in js], axis=2)
        h = (a1t + a2g).reshape(a_rows, K * d)
        sil = h + h * jnp.tanh(h)
        g4 = jnp.dot(sil, wg2k, preferred_element_type=jnp.float32) + bg2k
        w = 3 * min(K, n - j0)
        for a in range(3):
            c0 = 3 * n * a + 3 * j0
            out_ref[:, c0:c0 + w] = g4[:, 3 * K * a:3 * K * a + w]


@functools.partial(jax.jit, static_argnames=("interpret",))
def _run(R, s_rep, v_rep, params, Z, interpret=False):
    p = params
    N = R.shape[0]
    n = _N_ATOMS
    d = _D
    B = N // n
    G = 64
    K = 12
    A = G * n

    v_flat = v_rep.reshape(N, 3 * d)
    onehot = (Z[:, None] == jnp.array([1, 6, 8], dtype=Z.dtype)[None, :]).astype(jnp.float32)
    rz = jnp.concatenate([R, onehot], axis=1)

    def row(b):
        return b.reshape(1, -1)

    weights = [
        jax.scipy.linalg.block_diag(*([p['Wmix1']] * 3)),
        p['W1a'], row(p['b1a']), p['W2a'], row(p['b2a']),
        p['Wmix2'], p['W1b'], row(p['b1b']), p['W2b'], row(p['b2b']),
        p['Wp1'], row(p['bp1']), p['Wp2'], row(p['bp2']),
        p['Wf1'], row(p['bf1']), p['Wf2'], row(p['bf2']),
        0.5 * p['Wg1'], 0.5 * row(p['bg1']),
        jax.scipy.linalg.block_diag(*([p['Wg2']] * K)).reshape(K * d, K, 3, 3)
            .transpose(0, 2, 1, 3).reshape(K * d, 9 * K),
        row(p['bg2']).reshape(1, 1, 3, 3).repeat(K, axis=1)
            .transpose(0, 2, 1, 3).reshape(1, 9 * K),
        jnp.concatenate([jnp.eye(d, dtype=jnp.float32)] * K, axis=1),
    ]

    full = [pl.BlockSpec(w.shape, lambda g, nd=w.ndim: (0,) * nd) for w in weights]

    out = pl.pallas_call(
        functools.partial(_hessian_kernel, g_mols=G, n=n, d=d, pair_k=K),
        grid=(B // G,),
        in_specs=[
            pl.BlockSpec((A, d), lambda g: (g, 0)),
            pl.BlockSpec((A, 3 * d), lambda g: (g, 0)),
            pl.BlockSpec((A, 6), lambda g: (g, 0)),
        ] + full,
        out_specs=pl.BlockSpec((A, 9 * n), lambda g: (g, 0)),
        out_shape=jax.ShapeDtypeStruct((N, 9 * n), jnp.float32),
        interpret=interpret,
    )(s_rep, v_flat, rz, *weights)

    return out.reshape(-1, 27)


def kernel(R, scalar_representation, vector_representation, params, Z, n_atoms):
    del n_atoms
    return _run(R, scalar_representation, vector_representation, params, Z)
